# baseline (speedup 1.0000x reference)
import functools

import jax
import jax.numpy as jnp
import numpy as np
from jax import lax
from jax.experimental import pallas as pl
from jax.experimental.pallas import tpu as pltpu
from jax.experimental.pallas import tpu_sc as plsc

DIM = 128
UNIT_DIM = 16
N_PRIOR = 4096
BATCH = 16384
N_CHUNKS = DIM // UNIT_DIM

_NUM_CORES = 2
_NUM_SUBCORES = 16
_NW = _NUM_CORES * _NUM_SUBCORES

_ROWS_PER_W = BATCH // _NW
_G_PER_W = _ROWS_PER_W * N_CHUNKS
_PRIOR_PER_T = N_PRIOR // _NUM_SUBCORES


def _compute_indices():
    rkey = jax.random.key(42)
    perm = jax.random.permutation(jax.random.fold_in(rkey, 0), BATCH)
    sel = perm[:N_PRIOR].astype(jnp.int32)
    per_chunk = []
    for c in range(N_CHUNKS):
        ck = jax.random.fold_in(rkey, c + 1)
        per_chunk.append(jax.random.randint(ck, (BATCH,), 0, N_PRIOR))
    slot = jnp.stack(per_chunk, axis=1)
    h = slot + jnp.arange(N_CHUNKS, dtype=jnp.int32)[None, :] * N_PRIOR
    h = h.reshape(_NW, _ROWS_PER_W, N_CHUNKS).transpose(0, 2, 1)
    return sel, h.reshape(-1).astype(jnp.int32)


try:
    _SEL, _H = (np.asarray(a) for a in _compute_indices())
except Exception:
    _SEL = _H = None


def _gather_body(table_hbm, sel_hbm, h_hbm, out_hbm,
                 prior_v, sel_v, h_v, buf0_v, buf1_v, shared,
                 sem, sem_h, sem_s, sem_w):
    s = lax.axis_index("s")
    wid = s * _NUM_CORES + lax.axis_index("c")

    h_cp = pltpu.make_async_copy(
        h_hbm.at[pl.ds(wid * _G_PER_W, _G_PER_W)], h_v, sem_h)
    h_cp.start()

    pltpu.sync_copy(sel_hbm.at[pl.ds(s * _PRIOR_PER_T, _PRIOR_PER_T)], sel_v)
    pltpu.async_copy(table_hbm.at[sel_v], prior_v, sem).wait()
    spmem_cps = []
    for c in range(N_CHUNKS):
        cp = pltpu.make_async_copy(
            prior_v.at[:, pl.ds(c * UNIT_DIM, UNIT_DIM)],
            shared.at[pl.ds(c * N_PRIOR + s * _PRIOR_PER_T, _PRIOR_PER_T)],
            sem_s,
        )
        cp.start()
        spmem_cps.append(cp)
    for cp in spmem_cps:
        cp.wait()
    h_cp.wait()
    plsc.subcore_barrier()

    n_q = N_CHUNKS // 2
    q_rows = 2 * _ROWS_PER_W
    bufs = (buf0_v, buf1_v)
    g_cps = [None] * n_q
    w_cps = []

    def start_gather(q):
        cp = pltpu.make_async_copy(
            shared.at[h_v.at[pl.ds(q * q_rows, q_rows)]],
            bufs[q % 2], sem,
        )
        cp.start()
        g_cps[q] = cp

    def start_writes(q):
        for i in range(2):
            c = 2 * q + i
            cp = pltpu.make_async_copy(
                bufs[q % 2].at[pl.ds(i * _ROWS_PER_W, _ROWS_PER_W)],
                out_hbm.at[pl.ds(wid * _ROWS_PER_W, _ROWS_PER_W),
                           pl.ds(c * UNIT_DIM, UNIT_DIM)],
                sem_w,
            )
            cp.start()
            w_cps.append((q, cp))

    start_gather(0)
    for q in range(n_q):
        g_cps[q].wait()
        start_writes(q)
        if q + 1 < n_q:
            if q >= 1:
                for qq, cp in w_cps:
                    if qq == q - 1:
                        cp.wait()
            start_gather(q + 1)
    for qq, cp in w_cps:
        if qq >= n_q - 2:
            cp.wait()


@functools.cache
def _sc_gather():
    return pl.kernel(
        _gather_body,
        out_type=jax.ShapeDtypeStruct((BATCH, DIM), jnp.float32),
        compiler_params=pltpu.CompilerParams(use_tc_tiling_on_sc=False),
        mesh=plsc.VectorSubcoreMesh(
            core_axis_name="c",
            subcore_axis_name="s",
            num_cores=_NUM_CORES,
            num_subcores=_NUM_SUBCORES,
        ),
        scratch_types=[
            pltpu.VMEM((_PRIOR_PER_T, DIM), jnp.float32),
            pltpu.VMEM((_PRIOR_PER_T,), jnp.int32),
            pltpu.VMEM((_G_PER_W,), jnp.int32),
            pltpu.VMEM((2 * _ROWS_PER_W, UNIT_DIM), jnp.float32),
            pltpu.VMEM((2 * _ROWS_PER_W, UNIT_DIM), jnp.float32),
            pltpu.VMEM_SHARED((N_CHUNKS * N_PRIOR, UNIT_DIM), jnp.float32),
            pltpu.SemaphoreType.DMA,
            pltpu.SemaphoreType.DMA,
            pltpu.SemaphoreType.DMA,
            pltpu.SemaphoreType.DMA,
        ],
    )


def kernel(inputs):
    if _SEL is not None:
        sel, h = jnp.asarray(_SEL), jnp.asarray(_H)
    else:
        sel, h = _compute_indices()
    return _sc_gather()(inputs, sel, h)


# speedup vs baseline: 1.1408x; 1.1408x over previous
import functools

import jax
import jax.numpy as jnp
import numpy as np
from jax import lax
from jax.experimental import pallas as pl
from jax.experimental.pallas import tpu as pltpu
from jax.experimental.pallas import tpu_sc as plsc

DIM = 128
UNIT_DIM = 16
N_PRIOR = 4096
BATCH = 16384
N_CHUNKS = DIM // UNIT_DIM
N_ROWS = BATCH * N_CHUNKS

_NUM_CORES = 2
_NUM_SUBCORES = 16
_NW = _NUM_CORES * _NUM_SUBCORES

_ROWS_PER_W = BATCH // _NW
_G_PER_W = _ROWS_PER_W * N_CHUNKS
_N_Q = 4
_Q_ROWS = _G_PER_W // _N_Q


def _compute_indices():
    rkey = jax.random.key(42)
    perm = jax.random.permutation(jax.random.fold_in(rkey, 0), BATCH)
    sel = perm[:N_PRIOR]
    per_chunk = []
    for c in range(N_CHUNKS):
        ck = jax.random.fold_in(rkey, c + 1)
        per_chunk.append(jax.random.randint(ck, (BATCH,), 0, N_PRIOR))
    slot = jnp.stack(per_chunk, axis=1)
    fid = jnp.take(sel, slot, axis=0)
    g = fid * N_CHUNKS + jnp.arange(N_CHUNKS, dtype=jnp.int32)[None, :]
    g = g.reshape(_NW, _ROWS_PER_W, N_CHUNKS).transpose(0, 2, 1)
    return g.reshape(-1).astype(jnp.int32)


try:
    _G = np.asarray(_compute_indices())
except Exception:
    _G = None


def _gather_body(table_hbm, g_hbm, out_hbm,
                 g_v, buf0_v, buf1_v, sem_g, sem, sem_w):
    wid = lax.axis_index("s") * _NUM_CORES + lax.axis_index("c")
    pltpu.async_copy(g_hbm.at[pl.ds(wid * _G_PER_W, _G_PER_W)], g_v, sem_g
                     ).wait()

    bufs = (buf0_v, buf1_v)
    g_cps = [None] * _N_Q
    w_cps = []

    def start_gather(q):
        cp = pltpu.make_async_copy(
            table_hbm.at[g_v.at[pl.ds(q * _Q_ROWS, _Q_ROWS)]],
            bufs[q % 2], sem,
        )
        cp.start()
        g_cps[q] = cp

    def start_writes(q):
        for i in range(_Q_ROWS // _ROWS_PER_W):
            c = (_Q_ROWS // _ROWS_PER_W) * q + i
            cp = pltpu.make_async_copy(
                bufs[q % 2].at[pl.ds(i * _ROWS_PER_W, _ROWS_PER_W)],
                out_hbm.at[pl.ds(wid * _ROWS_PER_W, _ROWS_PER_W),
                           pl.ds(c * UNIT_DIM, UNIT_DIM)],
                sem_w,
            )
            cp.start()
            w_cps.append((q, cp))

    start_gather(0)
    for q in range(_N_Q):
        g_cps[q].wait()
        start_writes(q)
        if q + 1 < _N_Q:
            if q >= 1:
                for qq, cp in w_cps:
                    if qq == q - 1:
                        cp.wait()
            start_gather(q + 1)
    for qq, cp in w_cps:
        if qq >= _N_Q - 2:
            cp.wait()


@functools.cache
def _sc_gather():
    return pl.kernel(
        _gather_body,
        out_type=jax.ShapeDtypeStruct((BATCH, DIM), jnp.float32),
        compiler_params=pltpu.CompilerParams(use_tc_tiling_on_sc=False),
        mesh=plsc.VectorSubcoreMesh(
            core_axis_name="c",
            subcore_axis_name="s",
            num_cores=_NUM_CORES,
            num_subcores=_NUM_SUBCORES,
        ),
        scratch_types=[
            pltpu.VMEM((_G_PER_W,), jnp.int32),
            pltpu.VMEM((_Q_ROWS, UNIT_DIM), jnp.float32),
            pltpu.VMEM((_Q_ROWS, UNIT_DIM), jnp.float32),
            pltpu.SemaphoreType.DMA,
            pltpu.SemaphoreType.DMA,
            pltpu.SemaphoreType.DMA,
        ],
    )


def kernel(inputs):
    g = jnp.asarray(_G) if _G is not None else _compute_indices()
    table = inputs.reshape(N_ROWS, UNIT_DIM)
    return _sc_gather()(table, g)


# speedup vs baseline: 1.1971x; 1.0493x over previous
import functools

import jax
import jax.numpy as jnp
import numpy as np
from jax import lax
from jax.experimental import pallas as pl
from jax.experimental.pallas import tpu as pltpu
from jax.experimental.pallas import tpu_sc as plsc

DIM = 128
UNIT_DIM = 16
N_PRIOR = 4096
BATCH = 16384
N_CHUNKS = DIM // UNIT_DIM
N_ROWS = BATCH * N_CHUNKS

_NUM_CORES = 2
_NUM_SUBCORES = 16
_NW = _NUM_CORES * _NUM_SUBCORES

_ROWS_PER_W = BATCH // _NW
_G_PER_W = _ROWS_PER_W * N_CHUNKS
_N_Q = 4
_Q_ROWS = _G_PER_W // _N_Q


def _compute_indices():
    rkey = jax.random.key(42)
    perm = jax.random.permutation(jax.random.fold_in(rkey, 0), BATCH)
    sel = perm[:N_PRIOR]
    per_chunk = []
    for c in range(N_CHUNKS):
        ck = jax.random.fold_in(rkey, c + 1)
        per_chunk.append(jax.random.randint(ck, (BATCH,), 0, N_PRIOR))
    slot = jnp.stack(per_chunk, axis=1)
    fid = jnp.take(sel, slot, axis=0)
    g = fid * N_CHUNKS + jnp.arange(N_CHUNKS, dtype=jnp.int32)[None, :]
    return g.reshape(-1).astype(jnp.int32)


try:
    _G = np.asarray(_compute_indices())
except Exception:
    _G = None


def _gather_body(table_hbm, g_hbm, out_hbm,
                 g_v, buf0_v, buf1_v, sem_g, sem, sem_w):
    wid = lax.axis_index("s") * _NUM_CORES + lax.axis_index("c")
    pltpu.async_copy(g_hbm.at[pl.ds(wid * _G_PER_W, _G_PER_W)], g_v, sem_g
                     ).wait()

    bufs = (buf0_v, buf1_v)
    g_cps = [None] * _N_Q
    w_cps = []

    def start_gather(q):
        cp = pltpu.make_async_copy(
            table_hbm.at[g_v.at[pl.ds(q * _Q_ROWS, _Q_ROWS)]],
            bufs[q % 2], sem,
        )
        cp.start()
        g_cps[q] = cp

    def start_writes(q):
        cp = pltpu.make_async_copy(
            bufs[q % 2],
            out_hbm.at[pl.ds(wid * _G_PER_W + q * _Q_ROWS, _Q_ROWS)],
            sem_w,
        )
        cp.start()
        w_cps.append((q, cp))

    start_gather(0)
    for q in range(_N_Q):
        g_cps[q].wait()
        start_writes(q)
        if q + 1 < _N_Q:
            if q >= 1:
                for qq, cp in w_cps:
                    if qq == q - 1:
                        cp.wait()
            start_gather(q + 1)
    for qq, cp in w_cps:
        if qq >= _N_Q - 2:
            cp.wait()


@functools.cache
def _sc_gather():
    return pl.kernel(
        _gather_body,
        out_type=jax.ShapeDtypeStruct((N_ROWS, UNIT_DIM), jnp.float32),
        compiler_params=pltpu.CompilerParams(use_tc_tiling_on_sc=False),
        mesh=plsc.VectorSubcoreMesh(
            core_axis_name="c",
            subcore_axis_name="s",
            num_cores=_NUM_CORES,
            num_subcores=_NUM_SUBCORES,
        ),
        scratch_types=[
            pltpu.VMEM((_G_PER_W,), jnp.int32),
            pltpu.VMEM((_Q_ROWS, UNIT_DIM), jnp.float32),
            pltpu.VMEM((_Q_ROWS, UNIT_DIM), jnp.float32),
            pltpu.SemaphoreType.DMA,
            pltpu.SemaphoreType.DMA,
            pltpu.SemaphoreType.DMA,
        ],
    )


def kernel(inputs):
    g = jnp.asarray(_G) if _G is not None else _compute_indices()
    table = inputs.reshape(N_ROWS, UNIT_DIM)
    return _sc_gather()(table, g).reshape(BATCH, DIM)


# speedup vs baseline: 1.2533x; 1.0470x over previous
import functools

import jax
import jax.numpy as jnp
import numpy as np
from jax import lax
from jax.experimental import pallas as pl
from jax.experimental.pallas import tpu as pltpu
from jax.experimental.pallas import tpu_sc as plsc

DIM = 128
UNIT_DIM = 16
N_PRIOR = 4096
BATCH = 16384
N_CHUNKS = DIM // UNIT_DIM
N_ROWS = BATCH * N_CHUNKS

_NUM_CORES = 2
_NUM_SUBCORES = 16
_NW = _NUM_CORES * _NUM_SUBCORES

_ROWS_PER_W = BATCH // _NW
_G_PER_W = _ROWS_PER_W * N_CHUNKS
_N_Q = 2
_Q_ROWS = _G_PER_W // _N_Q


def _compute_indices():
    rkey = jax.random.key(42)
    perm = jax.random.permutation(jax.random.fold_in(rkey, 0), BATCH)
    sel = perm[:N_PRIOR]
    per_chunk = []
    for c in range(N_CHUNKS):
        ck = jax.random.fold_in(rkey, c + 1)
        per_chunk.append(jax.random.randint(ck, (BATCH,), 0, N_PRIOR))
    slot = jnp.stack(per_chunk, axis=1)
    fid = jnp.take(sel, slot, axis=0)
    g = fid * N_CHUNKS + jnp.arange(N_CHUNKS, dtype=jnp.int32)[None, :]
    return g.reshape(-1).astype(jnp.int32)


try:
    _G = np.asarray(_compute_indices())
except Exception:
    _G = None


def _gather_body(table_hbm, g_hbm, out_hbm,
                 g_v, buf0_v, buf1_v, sem_g, sem, sem_w):
    wid = lax.axis_index("s") * _NUM_CORES + lax.axis_index("c")
    pltpu.async_copy(g_hbm.at[pl.ds(wid * _G_PER_W, _G_PER_W)], g_v, sem_g
                     ).wait()

    bufs = (buf0_v, buf1_v)
    g_cps = [None] * _N_Q
    w_cps = []

    def start_gather(q):
        cp = pltpu.make_async_copy(
            table_hbm.at[g_v.at[pl.ds(q * _Q_ROWS, _Q_ROWS)]],
            bufs[q % 2], sem,
        )
        cp.start()
        g_cps[q] = cp

    def start_writes(q):
        cp = pltpu.make_async_copy(
            bufs[q % 2],
            out_hbm.at[pl.ds(wid * _G_PER_W + q * _Q_ROWS, _Q_ROWS)],
            sem_w,
        )
        cp.start()
        w_cps.append((q, cp))

    start_gather(0)
    for q in range(_N_Q):
        g_cps[q].wait()
        start_writes(q)
        if q + 1 < _N_Q:
            if q >= 1:
                for qq, cp in w_cps:
                    if qq == q - 1:
                        cp.wait()
            start_gather(q + 1)
    for qq, cp in w_cps:
        if qq >= _N_Q - 2:
            cp.wait()


@functools.cache
def _sc_gather():
    return pl.kernel(
        _gather_body,
        out_type=jax.ShapeDtypeStruct((N_ROWS, UNIT_DIM), jnp.float32),
        compiler_params=pltpu.CompilerParams(use_tc_tiling_on_sc=False),
        mesh=plsc.VectorSubcoreMesh(
            core_axis_name="c",
            subcore_axis_name="s",
            num_cores=_NUM_CORES,
            num_subcores=_NUM_SUBCORES,
        ),
        scratch_types=[
            pltpu.VMEM((_G_PER_W,), jnp.int32),
            pltpu.VMEM((_Q_ROWS, UNIT_DIM), jnp.float32),
            pltpu.VMEM((_Q_ROWS, UNIT_DIM), jnp.float32),
            pltpu.SemaphoreType.DMA,
            pltpu.SemaphoreType.DMA,
            pltpu.SemaphoreType.DMA,
        ],
    )


def kernel(inputs):
    g = jnp.asarray(_G) if _G is not None else _compute_indices()
    table = inputs.reshape(N_ROWS, UNIT_DIM)
    return _sc_gather()(table, g).reshape(BATCH, DIM)


# speedup vs baseline: 1.2805x; 1.0217x over previous
import functools

import jax
import jax.numpy as jnp
import numpy as np
from jax import lax
from jax.experimental import pallas as pl
from jax.experimental.pallas import tpu as pltpu
from jax.experimental.pallas import tpu_sc as plsc

DIM = 128
UNIT_DIM = 16
N_PRIOR = 4096
BATCH = 16384
N_CHUNKS = DIM // UNIT_DIM
N_ROWS = BATCH * N_CHUNKS

_NUM_CORES = 2
_NUM_SUBCORES = 16
_NW = _NUM_CORES * _NUM_SUBCORES

_ROWS_PER_W = BATCH // _NW
_G_PER_W = _ROWS_PER_W * N_CHUNKS
_N_Q = 1
_Q_ROWS = _G_PER_W // _N_Q


def _compute_indices():
    rkey = jax.random.key(42)
    perm = jax.random.permutation(jax.random.fold_in(rkey, 0), BATCH)
    sel = perm[:N_PRIOR]
    per_chunk = []
    for c in range(N_CHUNKS):
        ck = jax.random.fold_in(rkey, c + 1)
        per_chunk.append(jax.random.randint(ck, (BATCH,), 0, N_PRIOR))
    slot = jnp.stack(per_chunk, axis=1)
    fid = jnp.take(sel, slot, axis=0)
    g = fid * N_CHUNKS + jnp.arange(N_CHUNKS, dtype=jnp.int32)[None, :]
    return g.reshape(-1).astype(jnp.int32)


try:
    _G = np.asarray(_compute_indices())
except Exception:
    _G = None


def _gather_body(table_hbm, g_hbm, out_hbm,
                 g_v, buf0_v, buf1_v, sem_g, sem, sem_w):
    wid = lax.axis_index("s") * _NUM_CORES + lax.axis_index("c")
    pltpu.async_copy(g_hbm.at[pl.ds(wid * _G_PER_W, _G_PER_W)], g_v, sem_g
                     ).wait()

    bufs = (buf0_v, buf1_v)
    g_cps = [None] * _N_Q
    w_cps = []

    def start_gather(q):
        cp = pltpu.make_async_copy(
            table_hbm.at[g_v.at[pl.ds(q * _Q_ROWS, _Q_ROWS)]],
            bufs[q % 2], sem,
        )
        cp.start()
        g_cps[q] = cp

    def start_writes(q):
        cp = pltpu.make_async_copy(
            bufs[q % 2],
            out_hbm.at[pl.ds(wid * _G_PER_W + q * _Q_ROWS, _Q_ROWS)],
            sem_w,
        )
        cp.start()
        w_cps.append((q, cp))

    start_gather(0)
    for q in range(_N_Q):
        g_cps[q].wait()
        start_writes(q)
        if q + 1 < _N_Q:
            if q >= 1:
                for qq, cp in w_cps:
                    if qq == q - 1:
                        cp.wait()
            start_gather(q + 1)
    for qq, cp in w_cps:
        if qq >= _N_Q - 2:
            cp.wait()


@functools.cache
def _sc_gather():
    return pl.kernel(
        _gather_body,
        out_type=jax.ShapeDtypeStruct((N_ROWS, UNIT_DIM), jnp.float32),
        compiler_params=pltpu.CompilerParams(use_tc_tiling_on_sc=False),
        mesh=plsc.VectorSubcoreMesh(
            core_axis_name="c",
            subcore_axis_name="s",
            num_cores=_NUM_CORES,
            num_subcores=_NUM_SUBCORES,
        ),
        scratch_types=[
            pltpu.VMEM((_G_PER_W,), jnp.int32),
            pltpu.VMEM((_Q_ROWS, UNIT_DIM), jnp.float32),
            pltpu.VMEM((_Q_ROWS, UNIT_DIM), jnp.float32),
            pltpu.SemaphoreType.DMA,
            pltpu.SemaphoreType.DMA,
            pltpu.SemaphoreType.DMA,
        ],
    )


def kernel(inputs):
    g = jnp.asarray(_G) if _G is not None else _compute_indices()
    table = inputs.reshape(N_ROWS, UNIT_DIM)
    return _sc_gather()(table, g).reshape(BATCH, DIM)


# speedup vs baseline: 1.3351x; 1.0426x over previous
import functools

import jax
import jax.numpy as jnp
import numpy as np
from jax import lax
from jax.experimental import pallas as pl
from jax.experimental.pallas import tpu as pltpu
from jax.experimental.pallas import tpu_sc as plsc

DIM = 128
UNIT_DIM = 16
N_PRIOR = 4096
BATCH = 16384
N_CHUNKS = DIM // UNIT_DIM
N_ROWS = BATCH * N_CHUNKS

_NUM_CORES = 2
_NUM_SUBCORES = 16
_NW = _NUM_CORES * _NUM_SUBCORES

_G_PER_W = N_ROWS // _NW
_P_PER_W = _G_PER_W // 2
_LANES = 16


def _compute_packed_indices():
    rkey = jax.random.key(42)
    perm = jax.random.permutation(jax.random.fold_in(rkey, 0), BATCH)
    sel = perm[:N_PRIOR]
    per_chunk = []
    for c in range(N_CHUNKS):
        ck = jax.random.fold_in(rkey, c + 1)
        per_chunk.append(jax.random.randint(ck, (BATCH,), 0, N_PRIOR))
    slot = jnp.stack(per_chunk, axis=1)
    fid = jnp.take(sel, slot, axis=0).reshape(-1)
    fid = fid.astype(jnp.int32).reshape(-1, 2, _LANES)
    return fid[:, 0, :] | (fid[:, 1, :] << 16)


try:
    _GP = np.asarray(_compute_packed_indices()).reshape(-1)
except Exception:
    _GP = None


def _gather_body(table_hbm, gp_hbm, out_hbm, gp_v, g_v, buf_v, sem_g, sem):
    wid = lax.axis_index("s") * _NUM_CORES + lax.axis_index("c")
    pltpu.async_copy(
        gp_hbm.at[pl.ds(wid * _P_PER_W, _P_PER_W)], gp_v, sem_g).wait()

    cpat = lax.iota(jnp.int32, _LANES) & 7
    mask = jnp.full((_LANES,), 0xFFFF, dtype=jnp.int32)
    for i in range(_P_PER_W // _LANES):
        v = gp_v[pl.ds(i * _LANES, _LANES)]
        lo = ((v & mask) << 3) | cpat
        hi = (lax.shift_right_logical(v, 16) << 3) | cpat
        g_v[pl.ds(2 * i * _LANES, _LANES)] = lo
        g_v[pl.ds((2 * i + 1) * _LANES, _LANES)] = hi

    pltpu.async_copy(table_hbm.at[g_v], buf_v, sem).wait()
    pltpu.sync_copy(buf_v, out_hbm.at[pl.ds(wid * _G_PER_W, _G_PER_W)])


@functools.cache
def _sc_gather():
    return pl.kernel(
        _gather_body,
        out_type=jax.ShapeDtypeStruct((N_ROWS, UNIT_DIM), jnp.float32),
        compiler_params=pltpu.CompilerParams(use_tc_tiling_on_sc=False),
        mesh=plsc.VectorSubcoreMesh(
            core_axis_name="c",
            subcore_axis_name="s",
            num_cores=_NUM_CORES,
            num_subcores=_NUM_SUBCORES,
        ),
        scratch_types=[
            pltpu.VMEM((_P_PER_W,), jnp.int32),
            pltpu.VMEM((_G_PER_W,), jnp.int32),
            pltpu.VMEM((_G_PER_W, UNIT_DIM), jnp.float32),
            pltpu.SemaphoreType.DMA,
            pltpu.SemaphoreType.DMA,
        ],
    )


def kernel(inputs):
    if _GP is not None:
        gp = jnp.asarray(_GP)
    else:
        gp = _compute_packed_indices().reshape(-1)
    table = inputs.reshape(N_ROWS, UNIT_DIM)
    return _sc_gather()(table, gp).reshape(BATCH, DIM)


# speedup vs baseline: 1.3398x; 1.0035x over previous
import functools

import jax
import jax.numpy as jnp
import numpy as np
from jax import lax
from jax.experimental import pallas as pl
from jax.experimental.pallas import tpu as pltpu
from jax.experimental.pallas import tpu_sc as plsc

DIM = 128
UNIT_DIM = 16
N_PRIOR = 4096
BATCH = 16384
N_CHUNKS = DIM // UNIT_DIM
N_ROWS = BATCH * N_CHUNKS

_NUM_CORES = 2
_NUM_SUBCORES = 16
_NW = _NUM_CORES * _NUM_SUBCORES

_G_PER_W = N_ROWS // _NW
_P_PER_W = _G_PER_W // 2
_LANES = 16


def _compute_packed_indices():
    rkey = jax.random.key(42)
    perm = jax.random.permutation(jax.random.fold_in(rkey, 0), BATCH)
    sel = perm[:N_PRIOR]
    per_chunk = []
    for c in range(N_CHUNKS):
        ck = jax.random.fold_in(rkey, c + 1)
        per_chunk.append(jax.random.randint(ck, (BATCH,), 0, N_PRIOR))
    slot = jnp.stack(per_chunk, axis=1)
    fid = jnp.take(sel, slot, axis=0).reshape(-1)
    fid = fid.astype(jnp.int32).reshape(-1, 2, _LANES)
    return fid[:, 0, :] | (fid[:, 1, :] << 16)


try:
    _GP = np.asarray(_compute_packed_indices()).reshape(-1)
except Exception:
    _GP = None


def _gather_body(table_hbm, gp_hbm, out_hbm, gp_v, g_v, buf_v, sem_g, sem):
    wid = lax.axis_index("s") * _NUM_CORES + lax.axis_index("c")
    pltpu.async_copy(
        gp_hbm.at[pl.ds(wid * _P_PER_W, _P_PER_W)], gp_v, sem_g).wait()

    cpat = lax.iota(jnp.int32, _LANES) & 7
    mask = jnp.full((_LANES,), 0xFFFF, dtype=jnp.int32)

    def widen(half):
        base = half * (_P_PER_W // 2) // _LANES
        for i in range(base, base + (_P_PER_W // 2) // _LANES):
            v = gp_v[pl.ds(i * _LANES, _LANES)]
            lo = ((v & mask) << 3) | cpat
            hi = (lax.shift_right_logical(v, 16) << 3) | cpat
            g_v[pl.ds(2 * i * _LANES, _LANES)] = lo
            g_v[pl.ds((2 * i + 1) * _LANES, _LANES)] = hi

    half_g = _G_PER_W // 2

    def gather(half):
        cp = pltpu.make_async_copy(
            table_hbm.at[g_v.at[pl.ds(half * half_g, half_g)]],
            buf_v.at[pl.ds(half * half_g, half_g)], sem,
        )
        cp.start()
        return cp

    def write(half, w_sem):
        cp = pltpu.make_async_copy(
            buf_v.at[pl.ds(half * half_g, half_g)],
            out_hbm.at[pl.ds(wid * _G_PER_W + half * half_g, half_g)],
            w_sem,
        )
        cp.start()
        return cp

    widen(0)
    g0 = gather(0)
    widen(1)
    g1 = gather(1)
    g0.wait()
    w0 = write(0, sem_g)
    g1.wait()
    w1 = write(1, sem)
    w0.wait()
    w1.wait()


@functools.cache
def _sc_gather():
    return pl.kernel(
        _gather_body,
        out_type=jax.ShapeDtypeStruct((N_ROWS, UNIT_DIM), jnp.float32),
        compiler_params=pltpu.CompilerParams(use_tc_tiling_on_sc=False),
        mesh=plsc.VectorSubcoreMesh(
            core_axis_name="c",
            subcore_axis_name="s",
            num_cores=_NUM_CORES,
            num_subcores=_NUM_SUBCORES,
        ),
        scratch_types=[
            pltpu.VMEM((_P_PER_W,), jnp.int32),
            pltpu.VMEM((_G_PER_W,), jnp.int32),
            pltpu.VMEM((_G_PER_W, UNIT_DIM), jnp.float32),
            pltpu.SemaphoreType.DMA,
            pltpu.SemaphoreType.DMA,
        ],
    )


def kernel(inputs):
    if _GP is not None:
        gp = jnp.asarray(_GP)
    else:
        gp = _compute_packed_indices().reshape(-1)
    table = inputs.reshape(N_ROWS, UNIT_DIM)
    return _sc_gather()(table, gp).reshape(BATCH, DIM)
